# baseline (speedup 1.0000x reference)
import functools
import jax
import jax.numpy as jnp
from jax import lax
from jax.experimental import pallas as pl
from jax.experimental.pallas import tpu as pltpu

H = 128
BN = 1024


def _embed_body(x_ref, w_ref, b_ref, o_ref):
    o_ref[...] = (
        jnp.dot(x_ref[...], w_ref[...], preferred_element_type=jnp.float32)
        + b_ref[...]
    )


def _embed(x, w, b):
    n = x.shape[0]
    npad = ((n + BN - 1) // BN) * BN
    xp = jnp.pad(x, ((0, npad - n), (0, 0)))
    out = pl.pallas_call(
        _embed_body,
        grid=(npad // BN,),
        in_specs=[
            pl.BlockSpec((BN, x.shape[1]), lambda i: (i, 0)),
            pl.BlockSpec((x.shape[1], H), lambda i: (0, 0)),
            pl.BlockSpec((1, H), lambda i: (0, 0)),
        ],
        out_specs=pl.BlockSpec((BN, H), lambda i: (i, 0)),
        out_shape=jax.ShapeDtypeStruct((npad, H), jnp.float32),
    )(xp, w, b.reshape(1, H))
    return out[:n]


def _msg_body(x_ref, w_ref, b_ref, aw_ref, ab_ref, pm_ref, s_ref):
    pm = (
        jnp.dot(x_ref[...], w_ref[...], preferred_element_type=jnp.float32)
        + b_ref[...]
    )
    pm_ref[...] = pm
    s_ref[...] = (
        jnp.dot(pm, aw_ref[...], preferred_element_type=jnp.float32) + ab_ref[...]
    )


def _msg_and_score(x, w, b, aw, ab):
    npad = x.shape[0]
    return pl.pallas_call(
        _msg_body,
        grid=(npad // BN,),
        in_specs=[
            pl.BlockSpec((BN, H), lambda i: (i, 0)),
            pl.BlockSpec((H, H), lambda i: (0, 0)),
            pl.BlockSpec((1, H), lambda i: (0, 0)),
            pl.BlockSpec((H, 1), lambda i: (0, 0)),
            pl.BlockSpec((1, 1), lambda i: (0, 0)),
        ],
        out_specs=[
            pl.BlockSpec((BN, H), lambda i: (i, 0)),
            pl.BlockSpec((BN, 1), lambda i: (i, 0)),
        ],
        out_shape=[
            jax.ShapeDtypeStruct((npad, H), jnp.float32),
            jax.ShapeDtypeStruct((npad, 1), jnp.float32),
        ],
    )(x, w, b.reshape(1, H), aw, ab.reshape(1, 1))


def _update_body(x_ref, m_ref, w1_ref, w2_ref, b_ref, o_ref):
    xn = (
        jnp.dot(x_ref[...], w1_ref[...], preferred_element_type=jnp.float32)
        + jnp.dot(m_ref[...], w2_ref[...], preferred_element_type=jnp.float32)
        + b_ref[...]
    )
    o_ref[...] = jnp.maximum(x_ref[...] + xn, 0.0)


def _update(x, msgs, w1, w2, b):
    npad = x.shape[0]
    return pl.pallas_call(
        _update_body,
        grid=(npad // BN,),
        in_specs=[
            pl.BlockSpec((BN, H), lambda i: (i, 0)),
            pl.BlockSpec((BN, H), lambda i: (i, 0)),
            pl.BlockSpec((H, H), lambda i: (0, 0)),
            pl.BlockSpec((H, H), lambda i: (0, 0)),
            pl.BlockSpec((1, H), lambda i: (0, 0)),
        ],
        out_specs=pl.BlockSpec((BN, H), lambda i: (i, 0)),
        out_shape=jax.ShapeDtypeStruct((npad, H), jnp.float32),
    )(x, msgs, w1, w2, b.reshape(1, H))


def _edge_softmax_weights(s, cnt):
    s = s[:, 0]
    has = cnt > 0
    m = jnp.max(jnp.where(has, s, -jnp.inf))
    e = jnp.where(has, jnp.exp(s - m), 0.0)
    z = jnp.sum(cnt.astype(jnp.float32) * e)
    return e / z


def _messages(q, src, dst, n_dst):
    rows = jnp.take(q, src, axis=0)
    return jnp.zeros((n_dst, H), jnp.float32).at[dst].add(rows)


def kernel(place_features, transition_features, pre_edge_index, post_edge_index,
           place_emb_W, place_emb_b, trans_emb_W, trans_emb_b,
           p2t_W, p2t_b, t2p_W, t2p_b, pu_W, pu_b, tu_W, tu_b,
           pa_W, pa_b, ta_W, ta_b):
    n_place = place_features.shape[0]
    n_trans = transition_features.shape[0]
    n_layers = p2t_W.shape[0]
    np_pad = ((n_place + BN - 1) // BN) * BN
    nt_pad = ((n_trans + BN - 1) // BN) * BN

    p = _embed(place_features, place_emb_W, place_emb_b)
    t = _embed(transition_features, trans_emb_W, trans_emb_b)
    p = jnp.pad(p, ((0, np_pad - n_place), (0, 0)))
    t = jnp.pad(t, ((0, nt_pad - n_trans), (0, 0)))

    pi = pre_edge_index[0]
    ti = pre_edge_index[1]
    ti2 = post_edge_index[0]
    pi2 = post_edge_index[1]
    cnt_pre = jnp.zeros((np_pad,), jnp.int32).at[pi].add(1)
    cnt_post = jnp.zeros((nt_pad,), jnp.int32).at[ti2].add(1)

    for l in range(n_layers):
        pm, s_pre = _msg_and_score(p, p2t_W[l], p2t_b[l], ta_W[l], ta_b[l])
        tm, s_post = _msg_and_score(t, t2p_W[l], t2p_b[l], pa_W[l], pa_b[l])
        w_pre = _edge_softmax_weights(s_pre, cnt_pre)
        w_post = _edge_softmax_weights(s_post, cnt_post)
        q_pre = pm * w_pre[:, None]
        q_post = tm * w_post[:, None]
        trans_msg = _messages(q_pre, pi, ti, nt_pad)
        place_msg = _messages(q_post, ti2, pi2, np_pad)
        p = _update(p, place_msg, pu_W[l, :H], pu_W[l, H:], pu_b[l])
        t = _update(t, trans_msg, tu_W[l, :H], tu_W[l, H:], tu_b[l])

    return (p[:n_place], t[:n_trans])
